# baseline (speedup 1.0000x reference)
import dataclasses
import functools

import jax
import jax.numpy as jnp
from jax import lax
from jax.experimental import pallas as pl
from jax.experimental.pallas import tpu as pltpu
from jax.experimental.pallas import tpu_sc as plsc

_B0, _B1 = 16384, 50
_D = 32
_W = 256
_NC = _B0 // _W
_PACK = 128 // _D


def _sc_compiler_params():
    cp = pltpu.CompilerParams()
    if "needs_layout_passes" in pltpu.CompilerParams.__dataclass_fields__:
        cp = dataclasses.replace(cp, needs_layout_passes=False)
    return cp


def kernel(indices, weight):
    w4 = weight.reshape(1000000 // _PACK, 128)
    i_t = indices.T

    mesh = plsc.VectorSubcoreMesh(
        core_axis_name="core", subcore_axis_name="subcore"
    )

    @functools.partial(
        pl.kernel,
        out_type=jax.ShapeDtypeStruct((_B1, _D, _B0), weight.dtype),
        mesh=mesh,
        scratch_types=[
            pltpu.VMEM((_W, 128), jnp.float32),
            pltpu.VMEM((_W,), jnp.int32),
            pltpu.SemaphoreType.DMA,
        ],
        compiler_params=_sc_compiler_params(),
    )
    def k(w_hbm, i_hbm, o_hbm, g_v, p_v, sem):
        def body(i_vmem, o_vmem):
            for c in range(_W // 16):
                v = i_vmem[0, pl.ds(c * 16, 16)]
                p_v[pl.ds(c * 16, 16)] = v >> 2
            descs = [
                pltpu.async_copy(
                    w_hbm.at[p_v.at[pl.ds(k * 32, 32)]],
                    g_v.at[pl.ds(k * 32, 32)],
                    sem,
                )
                for k in range(_W // 32)
            ]
            for d in descs:
                d.wait()
            @plsc.parallel_loop(0, _W // 16, unroll=2)
            def _(c):
                v = i_vmem[0, pl.ds(c * 16, 16)]
                cb = (v & 3) * _D
                row = lax.iota(jnp.int32, 16) + c * 16
                for f in range(_D):
                    o_vmem[0, f, pl.ds(c * 16, 16)] = plsc.load_gather(
                        g_v, [row, cb + f]
                    )

        pltpu.emit_pipeline(
            body,
            grid=(_B1 * _NC,),
            in_specs=[
                pl.BlockSpec((1, _W), index_map=lambda i: (i // _NC, i % _NC))
            ],
            out_specs=[
                pl.BlockSpec(
                    (1, _D, _W), index_map=lambda i: (i // _NC, 0, i % _NC)
                )
            ],
            core_axis_name=("core", "subcore"),
            dimension_semantics=(pltpu.PARALLEL,),
        )(i_hbm, o_hbm)

    out_t = k(w4, i_t)
    return out_t.transpose(2, 0, 1)


# speedup vs baseline: 1.0017x; 1.0017x over previous
import dataclasses
import functools

import jax
import jax.numpy as jnp
from jax import lax
from jax.experimental import pallas as pl
from jax.experimental.pallas import tpu as pltpu
from jax.experimental.pallas import tpu_sc as plsc

_B0, _B1 = 16384, 50
_D = 32
_W = 256
_NC = _B0 // _W
_PACK = 128 // _D


def _sc_compiler_params():
    cp = pltpu.CompilerParams()
    if "needs_layout_passes" in pltpu.CompilerParams.__dataclass_fields__:
        cp = dataclasses.replace(cp, needs_layout_passes=False)
    return cp


def kernel(indices, weight):
    w4 = weight.reshape(1000000 // _PACK, 128)
    i_t = indices.T

    mesh = plsc.VectorSubcoreMesh(
        core_axis_name="core", subcore_axis_name="subcore"
    )

    @functools.partial(
        pl.kernel,
        out_type=jax.ShapeDtypeStruct((_B1, _D, _B0), weight.dtype),
        mesh=mesh,
        scratch_types=[
            pltpu.VMEM((_W, 128), jnp.float32),
            pltpu.VMEM((_W,), jnp.int32),
            pltpu.SemaphoreType.DMA,
            pltpu.SemaphoreType.DMA,
        ],
        compiler_params=_sc_compiler_params(),
    )
    def k(w_hbm, i_hbm, o_hbm, g_v, p_v, sem0, sem1):
        _H = _W // 2

        def body(i_vmem, o_vmem):
            sems = (sem0, sem1)
            descs = [None, None]

            def fire(h):
                base = h * _H
                for c in range(_H // 16):
                    v = i_vmem[0, pl.ds(base + c * 16, 16)]
                    p_v[pl.ds(base + c * 16, 16)] = v >> 2
                descs[h] = pltpu.async_copy(
                    w_hbm.at[p_v.at[pl.ds(base, _H)]],
                    g_v.at[pl.ds(base, _H)],
                    sems[h],
                )

            def extract(h):
                base = h * _H

                @plsc.parallel_loop(0, _H // 16, unroll=2)
                def _(c):
                    v = i_vmem[0, pl.ds(base + c * 16, 16)]
                    cb = (v & 3) * _D
                    row = lax.iota(jnp.int32, 16) + (base + c * 16)
                    for f in range(_D):
                        o_vmem[0, f, pl.ds(base + c * 16, 16)] = (
                            plsc.load_gather(g_v, [row, cb + f])
                        )

            fire(0)
            fire(1)
            descs[0].wait()
            extract(0)
            descs[1].wait()
            extract(1)

        pltpu.emit_pipeline(
            body,
            grid=(_B1 * _NC,),
            in_specs=[
                pl.BlockSpec((1, _W), index_map=lambda i: (i // _NC, i % _NC))
            ],
            out_specs=[
                pl.BlockSpec(
                    (1, _D, _W), index_map=lambda i: (i // _NC, 0, i % _NC)
                )
            ],
            core_axis_name=("core", "subcore"),
            dimension_semantics=(pltpu.PARALLEL,),
        )(i_hbm, o_hbm)

    out_t = k(w4, i_t)
    return out_t.transpose(2, 0, 1)


# speedup vs baseline: 1.0635x; 1.0616x over previous
import functools

import jax
import jax.numpy as jnp
from jax.experimental import pallas as pl
from jax.experimental.pallas import tpu as pltpu
from jax.experimental.pallas import tpu_sc as plsc

_B0, _B1 = 16384, 50
_D = 32
_RB = 16


def kernel(indices, weight):
    mesh = plsc.VectorSubcoreMesh(
        core_axis_name="core", subcore_axis_name="subcore"
    )

    @functools.partial(
        pl.kernel,
        out_type=jax.ShapeDtypeStruct((_B0, _B1, _D), weight.dtype),
        mesh=mesh,
        scratch_types=[pltpu.SemaphoreType.DMA],
        compiler_params=pltpu.CompilerParams(use_tc_tiling_on_sc=False),
    )
    def k(w_hbm, i_hbm, o_hbm, sem):
        def body(i_vmem, o_vmem):
            descs = [
                pltpu.async_copy(w_hbm.at[i_vmem.at[j]], o_vmem.at[j], sem)
                for j in range(_RB)
            ]
            for d in descs:
                d.wait()

        pltpu.emit_pipeline(
            body,
            grid=(_B0 // _RB,),
            in_specs=[pl.BlockSpec((_RB, _B1), index_map=lambda i: (i, 0))],
            out_specs=[
                pl.BlockSpec((_RB, _B1, _D), index_map=lambda i: (i, 0, 0))
            ],
            core_axis_name=("core", "subcore"),
            dimension_semantics=(pltpu.PARALLEL,),
        )(i_hbm, o_hbm)

    return k(weight, indices)
